# baseline (speedup 1.0000x reference)
import functools

import jax
import jax.numpy as jnp
from jax import lax
from jax.experimental import pallas as pl
from jax.experimental.pallas import tpu as pltpu
from jax.experimental.pallas import tpu_sc as plsc

NC = 2
NS = 16
NW = NC * NS
LANES = 16


def _sc_segment_partials(table, gidx4, dst4, *, N, D, n_stages, n_stage, C):
    rows_per_tile = (N // NS) // 8 * 8
    tail_rows = N - rows_per_tile * NS
    mesh = plsc.VectorSubcoreMesh(core_axis_name="c", subcore_axis_name="s")

    @functools.partial(
        pl.kernel,
        out_type=jax.ShapeDtypeStruct((NC, N, D), jnp.float32),
        mesh=mesh,
        scratch_types=[
            pltpu.VMEM((2, n_stage, C), jnp.int32),
            pltpu.VMEM((2, n_stage, C), jnp.int32),
            pltpu.VMEM((C, D), jnp.float32),
            pltpu.VMEM((C, D), jnp.float32),
            pltpu.VMEM_SHARED((N, D), jnp.float32),
            pltpu.SemaphoreType.DMA,
            pltpu.SemaphoreType.DMA,
            pltpu.SemaphoreType.DMA,
            pltpu.SemaphoreType.DMA,
            pltpu.SemaphoreType.DMA,
            pltpu.SemaphoreType.DMA,
        ],
    )
    def sc_kernel(table_h, gidx_h, dst_h, out_h,
                  eb0, eb1, rows0_v, rows1_v, agg_s,
                  semg0, semg1, sems0, sems1, st0, st1):
        c = lax.axis_index("c")
        s = lax.axis_index("s")
        wid = c * NS + s
        row0 = s * rows_per_tile
        ebufs = (eb0, eb1)

        def stage_start(st, buf):
            d0 = pltpu.async_copy(gidx_h.at[wid, st], buf.at[0], st0)
            d1 = pltpu.async_copy(dst_h.at[wid, st], buf.at[1], st1)
            return d0, d1

        d0, d1 = stage_start(0, eb0)

        zv = jnp.zeros((LANES,), jnp.float32)

        def zfill_body(i, carry):
            for k in range(D // LANES):
                rows0_v[i, pl.ds(k * LANES, LANES)] = zv
            return carry
        lax.fori_loop(0, C, zfill_body, 0)

        n_zcopy = rows_per_tile // C
        z_rem = rows_per_tile - n_zcopy * C
        for k in range(n_zcopy):
            pltpu.sync_copy(rows0_v, agg_s.at[pl.ds(row0 + k * C, C)])
        if z_rem:
            pltpu.sync_copy(rows0_v.at[pl.ds(0, z_rem)],
                            agg_s.at[pl.ds(row0 + n_zcopy * C, z_rem)])
        if tail_rows:
            @pl.when(s == NS - 1)
            def _():
                pltpu.sync_copy(rows0_v.at[pl.ds(0, tail_rows)],
                                agg_s.at[pl.ds(NS * rows_per_tile, tail_rows)])
        d0.wait()
        d1.wait()

        plsc.subcore_barrier()

        def gather(e_v, j, rbuf, sem):
            return pltpu.async_copy(table_h.at[e_v.at[0, j]], rbuf, sem)

        def scatter(e_v, j, rbuf, sem):
            return pltpu.async_copy(rbuf, agg_s.at[e_v.at[1, j]], sem, add=True)

        def gather_wait(e_v, j, rbuf, sem):
            pltpu.make_async_copy(table_h.at[e_v.at[0, j]], rbuf, sem).wait()

        def scatter_wait(e_v, j, rbuf, sem):
            pltpu.make_async_copy(rbuf, agg_s.at[e_v.at[1, j]], sem).wait()

        for st in range(n_stages):
            e_v = ebufs[st & 1]
            if st + 1 < n_stages:
                p0, p1 = stage_start(st + 1, ebufs[1 - (st & 1)])

            gather(e_v, 0, rows0_v, semg0)
            gather(e_v, 1, rows1_v, semg1)
            gather_wait(e_v, 0, rows0_v, semg0)
            scatter(e_v, 0, rows0_v, sems0)

            def pair_body(p, carry2, e_v=e_v):
                jo = 2 * p + 1
                je = 2 * p + 2
                scatter_wait(e_v, je - 2, rows0_v, sems0)
                gather(e_v, je, rows0_v, semg0)
                gather_wait(e_v, jo, rows1_v, semg1)
                scatter(e_v, jo, rows1_v, sems1)
                scatter_wait(e_v, jo, rows1_v, sems1)
                gather(e_v, je + 1, rows1_v, semg1)
                gather_wait(e_v, je, rows0_v, semg0)
                scatter(e_v, je, rows0_v, sems0)
                return carry2
            n_pairs = (n_stage - 3) // 2
            lax.fori_loop(0, n_pairs, pair_body, 0)

            jo = n_stage - 2
            je = n_stage - 1
            scatter_wait(e_v, je - 2, rows0_v, sems0)
            gather(e_v, je, rows0_v, semg0)
            gather_wait(e_v, jo, rows1_v, semg1)
            scatter(e_v, jo, rows1_v, sems1)
            gather_wait(e_v, je, rows0_v, semg0)
            scatter(e_v, je, rows0_v, sems0)
            scatter_wait(e_v, jo, rows1_v, sems1)
            scatter_wait(e_v, je, rows0_v, sems0)

            if st + 1 < n_stages:
                p0.wait()
                p1.wait()

        plsc.subcore_barrier()

        pltpu.sync_copy(agg_s.at[pl.ds(row0, rows_per_tile)],
                        out_h.at[c, pl.ds(row0, rows_per_tile)])
        if tail_rows:
            @pl.when(s == NS - 1)
            def _():
                pltpu.sync_copy(agg_s.at[pl.ds(NS * rows_per_tile, tail_rows)],
                                out_h.at[c, pl.ds(NS * rows_per_tile, tail_rows)])

    return sc_kernel(table, gidx4, dst4)


def _tc_table(x, W_rel, *, N, D, R, BN=1000):
    def body(x_ref, w_ref, o_ref):
        xb = x_ref[...]
        for r in range(R):
            o_ref[:, r * D:(r + 1) * D] = jnp.dot(
                xb, w_ref[r], preferred_element_type=jnp.float32)

    return pl.pallas_call(
        body,
        grid=(N // BN,),
        in_specs=[
            pl.BlockSpec((BN, D), lambda i: (i, 0)),
            pl.BlockSpec((R, D, D), lambda i: (0, 0, 0)),
        ],
        out_specs=pl.BlockSpec((BN, R * D), lambda i: (i, 0)),
        out_shape=jax.ShapeDtypeStruct((N, R * D), jnp.float32),
    )(x, W_rel)


def _tc_final(x, partials, W_self, b_rel, W1a, W1b, b1, W2a, W2b, b2, *, N, D, BN=1000):
    def body(x_ref, p_ref, ws_ref, brel_ref, w1a_ref, w1b_ref, b1_ref,
             w2a_ref, w2b_ref, b2_ref, o_ref):
        xb = x_ref[...]
        agg = p_ref[0] + p_ref[1]
        h = agg + jnp.dot(xb, ws_ref[...], preferred_element_type=jnp.float32)
        msg = jnp.tanh(h + brel_ref[...])
        mid = jnp.tanh(
            jnp.dot(xb, w1a_ref[...], preferred_element_type=jnp.float32)
            + jnp.dot(msg, w1b_ref[...], preferred_element_type=jnp.float32)
            + b1_ref[...])
        o_ref[...] = jnp.tanh(
            jnp.dot(xb, w2a_ref[...], preferred_element_type=jnp.float32)
            + jnp.dot(mid, w2b_ref[...], preferred_element_type=jnp.float32)
            + b2_ref[...])

    def full(shape):
        return pl.BlockSpec(shape, lambda i: tuple(0 for _ in shape))

    return pl.pallas_call(
        body,
        grid=(N // BN,),
        in_specs=[
            pl.BlockSpec((BN, D), lambda i: (i, 0)),
            pl.BlockSpec((NC, BN, D), lambda i: (0, i, 0)),
            full((D, D)),
            full((1, D)),
            full((D, 2 * D)),
            full((D, 2 * D)),
            full((1, 2 * D)),
            full((D, D)),
            full((2 * D, D)),
            full((1, D)),
        ],
        out_specs=pl.BlockSpec((BN, D), lambda i: (i, 0)),
        out_shape=jax.ShapeDtypeStruct((N, D), jnp.float32),
    )(x, partials, W_self, b_rel, W1a, W1b, b1, W2a, W2b, b2)


def kernel(x, W_rel, W_self, b_rel, W1, b1, W2, b2, edge_index, edges_type,
           nodes_type, is_block):
    N, D = x.shape
    R = W_rel.shape[0]
    E = edges_type.shape[0]
    C = 80
    n_chunks = E // (NW * C)
    n_stages = 5
    n_stage = n_chunks // n_stages

    table = _tc_table(x, W_rel, N=N, D=D, R=R).reshape(N * R, D)

    gidx4 = (edge_index[0] * R + edges_type).reshape(NW, n_stages, n_stage, C)
    dst4 = edge_index[1].reshape(NW, n_stages, n_stage, C)

    partials = _sc_segment_partials(table, gidx4, dst4,
                                    N=N, D=D, n_stages=n_stages,
                                    n_stage=n_stage, C=C)

    return _tc_final(
        x, partials, W_self, b_rel.reshape(1, D),
        W1[:D], W1[D:], b1.reshape(1, 2 * D),
        W2[:D], W2[D:], b2.reshape(1, D),
        N=N, D=D)


# speedup vs baseline: 1.0045x; 1.0045x over previous
import functools

import jax
import jax.numpy as jnp
from jax import lax
from jax.experimental import pallas as pl
from jax.experimental.pallas import tpu as pltpu
from jax.experimental.pallas import tpu_sc as plsc

NC = 2
NS = 16
NW = NC * NS
LANES = 16


def _sc_segment_partials(table, gidx4, dst4, *, N, D, n_stages, n_stage, C):
    rows_per_tile = (N // NS) // 8 * 8
    tail_rows = N - rows_per_tile * NS
    mesh = plsc.VectorSubcoreMesh(core_axis_name="c", subcore_axis_name="s")

    @functools.partial(
        pl.kernel,
        out_type=jax.ShapeDtypeStruct((NC, N, D), jnp.float32),
        mesh=mesh,
        scratch_types=[
            pltpu.VMEM((2, n_stage, C), jnp.int32),
            pltpu.VMEM((2, n_stage, C), jnp.int32),
            pltpu.VMEM((C, D), jnp.float32),
            pltpu.VMEM((C, D), jnp.float32),
            pltpu.VMEM_SHARED((N, D), jnp.float32),
            pltpu.SemaphoreType.DMA,
            pltpu.SemaphoreType.DMA,
            pltpu.SemaphoreType.DMA,
            pltpu.SemaphoreType.DMA,
            pltpu.SemaphoreType.DMA,
            pltpu.SemaphoreType.DMA,
        ],
    )
    def sc_kernel(table_h, gidx_h, dst_h, out_h,
                  eb0, eb1, rows0_v, rows1_v, agg_s,
                  semg0, semg1, sems0, sems1, st0, st1):
        c = lax.axis_index("c")
        s = lax.axis_index("s")
        wid = c * NS + s
        row0 = s * rows_per_tile
        ebufs = (eb0, eb1)

        def stage_start(st, buf):
            d0 = pltpu.async_copy(gidx_h.at[wid, st], buf.at[0], st0)
            d1 = pltpu.async_copy(dst_h.at[wid, st], buf.at[1], st1)
            return d0, d1

        d0, d1 = stage_start(0, eb0)

        zv = jnp.zeros((LANES,), jnp.float32)

        def zfill_body(i, carry):
            for k in range(D // LANES):
                rows0_v[i, pl.ds(k * LANES, LANES)] = zv
            return carry
        lax.fori_loop(0, C, zfill_body, 0)

        n_zcopy = rows_per_tile // C
        z_rem = rows_per_tile - n_zcopy * C
        for k in range(n_zcopy):
            pltpu.sync_copy(rows0_v, agg_s.at[pl.ds(row0 + k * C, C)])
        if z_rem:
            pltpu.sync_copy(rows0_v.at[pl.ds(0, z_rem)],
                            agg_s.at[pl.ds(row0 + n_zcopy * C, z_rem)])
        if tail_rows:
            @pl.when(s == NS - 1)
            def _():
                pltpu.sync_copy(rows0_v.at[pl.ds(0, tail_rows)],
                                agg_s.at[pl.ds(NS * rows_per_tile, tail_rows)])
        d0.wait()
        d1.wait()

        plsc.subcore_barrier()

        def gather(e_v, j, rbuf, sem):
            return pltpu.async_copy(table_h.at[e_v.at[0, j]], rbuf, sem)

        def scatter(e_v, j, rbuf, sem):
            return pltpu.async_copy(rbuf, agg_s.at[e_v.at[1, j]], sem, add=True)

        def gather_wait(e_v, j, rbuf, sem):
            pltpu.make_async_copy(table_h.at[e_v.at[0, j]], rbuf, sem).wait()

        def scatter_wait(e_v, j, rbuf, sem):
            pltpu.make_async_copy(rbuf, agg_s.at[e_v.at[1, j]], sem).wait()

        for st in range(n_stages):
            e_v = ebufs[st & 1]
            if st + 1 < n_stages:
                p0, p1 = stage_start(st + 1, ebufs[1 - (st & 1)])

            gather(e_v, 0, rows0_v, semg0)
            gather(e_v, 1, rows1_v, semg1)
            gather_wait(e_v, 0, rows0_v, semg0)
            scatter(e_v, 0, rows0_v, sems0)

            def pair_body(p, carry2, e_v=e_v):
                jo = 2 * p + 1
                je = 2 * p + 2
                scatter_wait(e_v, je - 2, rows0_v, sems0)
                gather(e_v, je, rows0_v, semg0)
                gather_wait(e_v, jo, rows1_v, semg1)
                scatter(e_v, jo, rows1_v, sems1)
                scatter_wait(e_v, jo, rows1_v, sems1)
                gather(e_v, je + 1, rows1_v, semg1)
                gather_wait(e_v, je, rows0_v, semg0)
                scatter(e_v, je, rows0_v, sems0)
                return carry2
            n_pairs = (n_stage - 3) // 2
            lax.fori_loop(0, n_pairs, pair_body, 0)

            jo = n_stage - 2
            je = n_stage - 1
            scatter_wait(e_v, je - 2, rows0_v, sems0)
            gather(e_v, je, rows0_v, semg0)
            gather_wait(e_v, jo, rows1_v, semg1)
            scatter(e_v, jo, rows1_v, sems1)
            gather_wait(e_v, je, rows0_v, semg0)
            scatter(e_v, je, rows0_v, sems0)
            scatter_wait(e_v, jo, rows1_v, sems1)
            scatter_wait(e_v, je, rows0_v, sems0)

            if st + 1 < n_stages:
                p0.wait()
                p1.wait()

        plsc.subcore_barrier()

        pltpu.sync_copy(agg_s.at[pl.ds(row0, rows_per_tile)],
                        out_h.at[c, pl.ds(row0, rows_per_tile)])
        if tail_rows:
            @pl.when(s == NS - 1)
            def _():
                pltpu.sync_copy(agg_s.at[pl.ds(NS * rows_per_tile, tail_rows)],
                                out_h.at[c, pl.ds(NS * rows_per_tile, tail_rows)])

    return sc_kernel(table, gidx4, dst4)


def _tc_table(x, W_rel, *, N, D, R, BN=5000):
    def body(x_ref, w_ref, o_ref):
        xb = x_ref[...]
        for r in range(R):
            o_ref[:, r * D:(r + 1) * D] = jnp.dot(
                xb, w_ref[r], preferred_element_type=jnp.float32)

    return pl.pallas_call(
        body,
        grid=(N // BN,),
        in_specs=[
            pl.BlockSpec((BN, D), lambda i: (i, 0)),
            pl.BlockSpec((R, D, D), lambda i: (0, 0, 0)),
        ],
        out_specs=pl.BlockSpec((BN, R * D), lambda i: (i, 0)),
        out_shape=jax.ShapeDtypeStruct((N, R * D), jnp.float32),
    )(x, W_rel)


def _tc_final(x, partials, W_self, b_rel, W1a, W1b, b1, W2a, W2b, b2, *, N, D, BN=5000):
    def body(x_ref, p_ref, ws_ref, brel_ref, w1a_ref, w1b_ref, b1_ref,
             w2a_ref, w2b_ref, b2_ref, o_ref):
        xb = x_ref[...]
        agg = p_ref[0] + p_ref[1]
        h = agg + jnp.dot(xb, ws_ref[...], preferred_element_type=jnp.float32)
        msg = jnp.tanh(h + brel_ref[...])
        mid = jnp.tanh(
            jnp.dot(xb, w1a_ref[...], preferred_element_type=jnp.float32)
            + jnp.dot(msg, w1b_ref[...], preferred_element_type=jnp.float32)
            + b1_ref[...])
        o_ref[...] = jnp.tanh(
            jnp.dot(xb, w2a_ref[...], preferred_element_type=jnp.float32)
            + jnp.dot(mid, w2b_ref[...], preferred_element_type=jnp.float32)
            + b2_ref[...])

    def full(shape):
        return pl.BlockSpec(shape, lambda i: tuple(0 for _ in shape))

    return pl.pallas_call(
        body,
        grid=(N // BN,),
        in_specs=[
            pl.BlockSpec((BN, D), lambda i: (i, 0)),
            pl.BlockSpec((NC, BN, D), lambda i: (0, i, 0)),
            full((D, D)),
            full((1, D)),
            full((D, 2 * D)),
            full((D, 2 * D)),
            full((1, 2 * D)),
            full((D, D)),
            full((2 * D, D)),
            full((1, D)),
        ],
        out_specs=pl.BlockSpec((BN, D), lambda i: (i, 0)),
        out_shape=jax.ShapeDtypeStruct((N, D), jnp.float32),
    )(x, partials, W_self, b_rel, W1a, W1b, b1, W2a, W2b, b2)


def kernel(x, W_rel, W_self, b_rel, W1, b1, W2, b2, edge_index, edges_type,
           nodes_type, is_block):
    N, D = x.shape
    R = W_rel.shape[0]
    E = edges_type.shape[0]
    C = 80
    n_chunks = E // (NW * C)
    n_stages = 5
    n_stage = n_chunks // n_stages

    table = _tc_table(x, W_rel, N=N, D=D, R=R).reshape(N * R, D)

    gidx4 = (edge_index[0] * R + edges_type).reshape(NW, n_stages, n_stage, C)
    dst4 = edge_index[1].reshape(NW, n_stages, n_stage, C)

    partials = _sc_segment_partials(table, gidx4, dst4,
                                    N=N, D=D, n_stages=n_stages,
                                    n_stage=n_stage, C=C)

    return _tc_final(
        x, partials, W_self, b_rel.reshape(1, D),
        W1[:D], W1[D:], b1.reshape(1, 2 * D),
        W2[:D], W2[D:], b2.reshape(1, D),
        N=N, D=D)


# speedup vs baseline: 1.0316x; 1.0270x over previous
import functools

import jax
import jax.numpy as jnp
from jax import lax
from jax.experimental import pallas as pl
from jax.experimental.pallas import tpu as pltpu
from jax.experimental.pallas import tpu_sc as plsc

NC = 2
NS = 16
NW = NC * NS
LANES = 16


def _sc_segment_partials(table, gidx4, dst4, *, N, D, n_stages, n_stage, C):
    rows_per_tile = (N // NS) // 8 * 8
    tail_rows = N - rows_per_tile * NS
    mesh = plsc.VectorSubcoreMesh(core_axis_name="c", subcore_axis_name="s")

    @functools.partial(
        pl.kernel,
        out_type=jax.ShapeDtypeStruct((NC, N, D), jnp.float32),
        mesh=mesh,
        scratch_types=[
            pltpu.VMEM((2, n_stage, C), jnp.int32),
            pltpu.VMEM((2, n_stage, C), jnp.int32),
            pltpu.VMEM((C, D), jnp.float32),
            pltpu.VMEM((C, D), jnp.float32),
            pltpu.VMEM_SHARED((N, D), jnp.float32),
            pltpu.SemaphoreType.DMA,
            pltpu.SemaphoreType.DMA,
            pltpu.SemaphoreType.DMA,
            pltpu.SemaphoreType.DMA,
            pltpu.SemaphoreType.DMA,
            pltpu.SemaphoreType.DMA,
        ],
    )
    def sc_kernel(table_h, gidx_h, dst_h, out_h,
                  eb0, eb1, rows0_v, rows1_v, agg_s,
                  semg0, semg1, sems0, sems1, st0, st1):
        c = lax.axis_index("c")
        s = lax.axis_index("s")
        wid = c * NS + s
        row0 = s * rows_per_tile
        ebufs = (eb0, eb1)

        def stage_start(st, buf):
            d0 = pltpu.async_copy(gidx_h.at[wid, st], buf.at[0], st0)
            d1 = pltpu.async_copy(dst_h.at[wid, st], buf.at[1], st1)
            return d0, d1

        d0, d1 = stage_start(0, eb0)

        zv = jnp.zeros((LANES,), jnp.float32)

        def zfill_body(i, carry):
            for k in range(D // LANES):
                rows0_v[i, pl.ds(k * LANES, LANES)] = zv
            return carry
        lax.fori_loop(0, C, zfill_body, 0)

        n_zcopy = rows_per_tile // C
        z_rem = rows_per_tile - n_zcopy * C
        for k in range(n_zcopy):
            pltpu.sync_copy(rows0_v, agg_s.at[pl.ds(row0 + k * C, C)])
        if z_rem:
            pltpu.sync_copy(rows0_v.at[pl.ds(0, z_rem)],
                            agg_s.at[pl.ds(row0 + n_zcopy * C, z_rem)])
        if tail_rows:
            @pl.when(s == NS - 1)
            def _():
                pltpu.sync_copy(rows0_v.at[pl.ds(0, tail_rows)],
                                agg_s.at[pl.ds(NS * rows_per_tile, tail_rows)])
        d0.wait()
        d1.wait()

        plsc.subcore_barrier()

        def gather(e_v, j, rbuf, sem):
            return pltpu.async_copy(table_h.at[e_v.at[0, j]], rbuf, sem)

        def scatter(e_v, j, rbuf, sem):
            return pltpu.async_copy(rbuf, agg_s.at[e_v.at[1, j]], sem, add=True)

        def gather_wait(e_v, j, rbuf, sem):
            pltpu.make_async_copy(table_h.at[e_v.at[0, j]], rbuf, sem).wait()

        def scatter_wait(e_v, j, rbuf, sem):
            pltpu.make_async_copy(rbuf, agg_s.at[e_v.at[1, j]], sem).wait()

        for st in range(n_stages):
            e_v = ebufs[st & 1]
            if st + 1 < n_stages:
                p0, p1 = stage_start(st + 1, ebufs[1 - (st & 1)])

            gather(e_v, 0, rows0_v, semg0)
            gather(e_v, 1, rows1_v, semg1)
            gather_wait(e_v, 0, rows0_v, semg0)
            scatter(e_v, 0, rows0_v, sems0)

            def pair_body(p, carry2, e_v=e_v):
                jo = 2 * p + 1
                je = 2 * p + 2
                scatter_wait(e_v, je - 2, rows0_v, sems0)
                gather(e_v, je, rows0_v, semg0)
                gather_wait(e_v, jo, rows1_v, semg1)
                scatter(e_v, jo, rows1_v, sems1)
                scatter_wait(e_v, jo, rows1_v, sems1)
                gather(e_v, je + 1, rows1_v, semg1)
                gather_wait(e_v, je, rows0_v, semg0)
                scatter(e_v, je, rows0_v, sems0)
                return carry2
            n_pairs = (n_stage - 3) // 2
            lax.fori_loop(0, n_pairs, pair_body, 0)

            jo = n_stage - 2
            je = n_stage - 1
            scatter_wait(e_v, je - 2, rows0_v, sems0)
            gather(e_v, je, rows0_v, semg0)
            gather_wait(e_v, jo, rows1_v, semg1)
            scatter(e_v, jo, rows1_v, sems1)
            gather_wait(e_v, je, rows0_v, semg0)
            scatter(e_v, je, rows0_v, sems0)
            scatter_wait(e_v, jo, rows1_v, sems1)
            scatter_wait(e_v, je, rows0_v, sems0)

            if st + 1 < n_stages:
                p0.wait()
                p1.wait()

        plsc.subcore_barrier()

        pltpu.sync_copy(agg_s.at[pl.ds(row0, rows_per_tile)],
                        out_h.at[c, pl.ds(row0, rows_per_tile)])
        if tail_rows:
            @pl.when(s == NS - 1)
            def _():
                pltpu.sync_copy(agg_s.at[pl.ds(NS * rows_per_tile, tail_rows)],
                                out_h.at[c, pl.ds(NS * rows_per_tile, tail_rows)])

    return sc_kernel(table, gidx4, dst4)


def _tc_table(x, W_rel, *, N, D, R, BN=2000):
    def body(x_ref, w_ref, o_ref):
        xb = x_ref[...]
        for r in range(R):
            o_ref[:, r * D:(r + 1) * D] = jnp.dot(
                xb, w_ref[r], preferred_element_type=jnp.float32)

    return pl.pallas_call(
        body,
        grid=(N // BN,),
        in_specs=[
            pl.BlockSpec((BN, D), lambda i: (i, 0)),
            pl.BlockSpec((R, D, D), lambda i: (0, 0, 0)),
        ],
        out_specs=pl.BlockSpec((BN, R * D), lambda i: (i, 0)),
        out_shape=jax.ShapeDtypeStruct((N, R * D), jnp.float32),
    )(x, W_rel)


def _tc_final(x, partials, W_self, b_rel, W1a, W1b, b1, W2a, W2b, b2, *, N, D, BN=2000):
    def body(x_ref, p_ref, ws_ref, brel_ref, w1a_ref, w1b_ref, b1_ref,
             w2a_ref, w2b_ref, b2_ref, o_ref):
        xb = x_ref[...]
        agg = p_ref[0] + p_ref[1]
        h = agg + jnp.dot(xb, ws_ref[...], preferred_element_type=jnp.float32)
        msg = jnp.tanh(h + brel_ref[...])
        mid = jnp.tanh(
            jnp.dot(xb, w1a_ref[...], preferred_element_type=jnp.float32)
            + jnp.dot(msg, w1b_ref[...], preferred_element_type=jnp.float32)
            + b1_ref[...])
        o_ref[...] = jnp.tanh(
            jnp.dot(xb, w2a_ref[...], preferred_element_type=jnp.float32)
            + jnp.dot(mid, w2b_ref[...], preferred_element_type=jnp.float32)
            + b2_ref[...])

    def full(shape):
        return pl.BlockSpec(shape, lambda i: tuple(0 for _ in shape))

    return pl.pallas_call(
        body,
        grid=(N // BN,),
        in_specs=[
            pl.BlockSpec((BN, D), lambda i: (i, 0)),
            pl.BlockSpec((NC, BN, D), lambda i: (0, i, 0)),
            full((D, D)),
            full((1, D)),
            full((D, 2 * D)),
            full((D, 2 * D)),
            full((1, 2 * D)),
            full((D, D)),
            full((2 * D, D)),
            full((1, D)),
        ],
        out_specs=pl.BlockSpec((BN, D), lambda i: (i, 0)),
        out_shape=jax.ShapeDtypeStruct((N, D), jnp.float32),
    )(x, partials, W_self, b_rel, W1a, W1b, b1, W2a, W2b, b2)


def kernel(x, W_rel, W_self, b_rel, W1, b1, W2, b2, edge_index, edges_type,
           nodes_type, is_block):
    N, D = x.shape
    R = W_rel.shape[0]
    E = edges_type.shape[0]
    C = 80
    n_chunks = E // (NW * C)
    n_stages = 5
    n_stage = n_chunks // n_stages

    table = _tc_table(x, W_rel, N=N, D=D, R=R).reshape(N * R, D)

    gidx4 = (edge_index[0] * R + edges_type).reshape(NW, n_stages, n_stage, C)
    dst4 = edge_index[1].reshape(NW, n_stages, n_stage, C)

    partials = _sc_segment_partials(table, gidx4, dst4,
                                    N=N, D=D, n_stages=n_stages,
                                    n_stage=n_stage, C=C)

    return _tc_final(
        x, partials, W_self, b_rel.reshape(1, D),
        W1[:D], W1[D:], b1.reshape(1, 2 * D),
        W2[:D], W2[D:], b2.reshape(1, D),
        N=N, D=D)
